# baseline (speedup 1.0000x reference)
import functools

import jax
import jax.numpy as jnp
from jax import lax
from jax.experimental import pallas as pl
from jax.experimental.pallas import tpu as pltpu
from jax.experimental.pallas import tpu_sc as plsc

WINDOW_SIZE = 128
WL = 2 * WINDOW_SIZE + 1
TWO_STD_SQ = 2.0 * (WINDOW_SIZE / 2.0) ** 2

_DOT = functools.partial(
    jax.lax.dot_general,
    preferred_element_type=jnp.float32,
)

_NW = 32
_WLP = 288
_RPW = 8 * _WLP // _NW


def _k0_body(dec_ref, w1_ref, b1_ref, w2_ref, b2_ref, len_ref, idx_ref):
    x = dec_ref[...]
    g = _DOT(x, w1_ref[...], (((1,), (1,)), ((), ())))
    t1 = jnp.tanh(g + b1_ref[...])
    z = _DOT(t1, w2_ref[...], (((1,), (1,)), ((), ())))[:, 0:1]
    sig = jax.nn.sigmoid(z + b2_ref[0])
    st = jnp.round(len_ref[...] * sig).astype(jnp.int32)
    B = st.shape[0]
    iw = jax.lax.broadcasted_iota(jnp.int32, (B, _WLP), 1)
    ib = jax.lax.broadcasted_iota(jnp.int32, (B, _WLP), 0)
    idx_ref[...] = jnp.where(iw < WL, (st + iw) * B + ib, 0)


def _sc_gather_body(idx_hbm, enc_hbm, out_hbm, idx_v, rows_v, sem):
    wid = lax.axis_index("s") * 2 + lax.axis_index("c")
    base = wid * _RPW
    pltpu.sync_copy(idx_hbm.at[pl.ds(base, _RPW)], idx_v)
    pltpu.async_copy(enc_hbm.at[idx_v], rows_v, sem).wait()
    pltpu.sync_copy(rows_v, out_hbm.at[pl.ds(base, _RPW)])


def _mlp_body(len_ref, dec_ref, w1_ref, b1_ref, w2_ref, b2_ref,
              p_ref, ws_ref, ht_v, sem_ht):
    b = pl.program_id(0)
    B = pl.num_programs(0)

    def ht_copy(bb):
        return pltpu.make_async_copy(
            dec_ref.at[:, bb, :], ht_v.at[bb], sem_ht.at[bb])

    @pl.when(b == 0)
    def _():
        for bb in range(B):
            ht_copy(bb).start()

    ht_copy(b).wait()
    ht = ht_v[b]
    g = _DOT(ht, w1_ref[...], (((1,), (1,)), ((), ())))
    t1 = jnp.tanh(g + b1_ref[...])
    z = _DOT(t1, w2_ref[...], (((1,), (1,)), ((), ())))[:, 0:1]
    sig = jax.nn.sigmoid(z + b2_ref[0])
    length = len_ref[b].astype(jnp.float32)
    p = WINDOW_SIZE + length * sig
    p_ref[0] = p
    ws_ref[0] = jnp.round(p - WINDOW_SIZE).astype(jnp.int32)


def _att_body(len_ref, dec_ref, sel_ref, p_ref, ws_ref, out_ref,
              ht_v, out_v, sem_ht, sem_out):
    b = pl.program_id(0)
    nb = pl.num_programs(0)
    B = nb

    def ht_copy(bb):
        return pltpu.make_async_copy(
            dec_ref.at[:, bb, :], ht_v.at[bb], sem_ht.at[bb])

    def out_copy(bb):
        return pltpu.make_async_copy(
            out_v.at[bb % 2], out_ref.at[:, bb, :], sem_out.at[bb % 2])

    @pl.when(b == 0)
    def _():
        for bb in range(B):
            ht_copy(bb).start()

    ht_copy(b).wait()
    ht = ht_v[b]
    p = p_ref[0]
    ws = ws_ref[0]

    T = ht.shape[0]
    iw = jax.lax.broadcasted_iota(jnp.int32, (T, WL), 1)
    pos = ws.astype(jnp.float32) + iw.astype(jnp.float32)
    gauss = jnp.exp(-((pos - p) ** 2) / TWO_STD_SQ)

    sel = sel_ref[0][:WL]
    score = _DOT(ht, sel, (((1,), (1,)), ((), ())))
    left = iw < (WINDOW_SIZE - ws)
    right = iw >= (len_ref[b] + WINDOW_SIZE - ws)
    score = jnp.where(left | right, jnp.float32(1e-14), score)
    m = jnp.max(score, axis=1, keepdims=True)
    e = jnp.exp(score - m)
    a = (e / jnp.sum(e, axis=1, keepdims=True)) * gauss

    @pl.when(b >= 2)
    def _():
        out_copy(b - 2).wait()

    c = _DOT(a[:, :WL - 1], sel[:WL - 1],
             (((1,), (0,)), ((), ())))
    c = c + a[:, WL - 1:WL] * sel[WL - 1:WL, :]
    out_v[b % 2] = c
    out_copy(b).start()

    @pl.when(b == nb - 1)
    def _():
        out_copy(b - 1).wait()
        out_copy(b).wait()


def kernel(encoder_output, decoder_output, lengths, fc1_w, fc1_b, fc2_w,
           fc2_b, T, batch_size, output_weights):
    S, B, H = encoder_output.shape
    Tn = decoder_output.shape[0]
    H2 = fc1_w.shape[0]

    b1 = fc1_b.reshape(1, H2)
    w2 = jnp.zeros((8, H2), jnp.float32).at[0].set(fc2_w.reshape(H2))
    b2 = fc2_b.reshape(1)
    len_f = lengths.astype(jnp.float32).reshape(B, 1)

    idx = pl.pallas_call(
        _k0_body,
        out_shape=jax.ShapeDtypeStruct((B, _WLP), jnp.int32),
        in_specs=[
            pl.BlockSpec(memory_space=pltpu.VMEM),
            pl.BlockSpec(memory_space=pltpu.VMEM),
            pl.BlockSpec(memory_space=pltpu.VMEM),
            pl.BlockSpec(memory_space=pltpu.VMEM),
            pl.BlockSpec(memory_space=pltpu.SMEM),
            pl.BlockSpec(memory_space=pltpu.VMEM),
        ],
        out_specs=pl.BlockSpec(memory_space=pltpu.VMEM),
    )(decoder_output[Tn - 1], fc1_w, b1, w2, b2, len_f)

    enc2d = encoder_output.reshape(S * B, H)

    sc_gather = functools.partial(
        pl.kernel,
        out_type=jax.ShapeDtypeStruct((B * _WLP, H), jnp.float32),
        mesh=plsc.VectorSubcoreMesh(core_axis_name="c", subcore_axis_name="s"),
        scratch_types=[
            pltpu.VMEM((_RPW,), jnp.int32),
            pltpu.VMEM((_RPW, H), jnp.float32),
            pltpu.SemaphoreType.DMA,
        ],
    )(_sc_gather_body)
    sel_flat = sc_gather(idx.reshape(B * _WLP), enc2d)
    selection = sel_flat.reshape(B, _WLP, H)

    p_all, ws_all = pl.pallas_call(
        _mlp_body,
        grid=(B,),
        in_specs=[
            pl.BlockSpec(memory_space=pltpu.SMEM),
            pl.BlockSpec(memory_space=pl.ANY),
            pl.BlockSpec(memory_space=pltpu.VMEM),
            pl.BlockSpec(memory_space=pltpu.VMEM),
            pl.BlockSpec(memory_space=pltpu.VMEM),
            pl.BlockSpec(memory_space=pltpu.SMEM),
        ],
        out_specs=[
            pl.BlockSpec((1, Tn, 1), lambda b: (b, 0, 0)),
            pl.BlockSpec((1, Tn, 1), lambda b: (b, 0, 0)),
        ],
        out_shape=[
            jax.ShapeDtypeStruct((B, Tn, 1), jnp.float32),
            jax.ShapeDtypeStruct((B, Tn, 1), jnp.int32),
        ],
        scratch_shapes=[
            pltpu.VMEM((B, Tn, H), jnp.float32),
            pltpu.SemaphoreType.DMA((B,)),
        ],
    )(lengths, decoder_output, fc1_w, b1, w2, b2)

    out = pl.pallas_call(
        _att_body,
        grid=(B,),
        in_specs=[
            pl.BlockSpec(memory_space=pltpu.SMEM),
            pl.BlockSpec(memory_space=pl.ANY),
            pl.BlockSpec((1, _WLP, H), lambda b: (b, 0, 0)),
            pl.BlockSpec((1, Tn, 1), lambda b: (b, 0, 0)),
            pl.BlockSpec((1, Tn, 1), lambda b: (b, 0, 0)),
        ],
        out_specs=pl.BlockSpec(memory_space=pl.ANY),
        out_shape=jax.ShapeDtypeStruct((Tn, B, H), jnp.float32),
        scratch_shapes=[
            pltpu.VMEM((B, Tn, H), jnp.float32),
            pltpu.VMEM((2, Tn, H), jnp.float32),
            pltpu.SemaphoreType.DMA((B,)),
            pltpu.SemaphoreType.DMA((2,)),
        ],
    )(lengths, decoder_output, selection, p_all, ws_all)
    return out


# speedup vs baseline: 1.0868x; 1.0868x over previous
import functools

import jax
import jax.numpy as jnp
from jax import lax
from jax.experimental import pallas as pl
from jax.experimental.pallas import tpu as pltpu
from jax.experimental.pallas import tpu_sc as plsc

WINDOW_SIZE = 128
WL = 2 * WINDOW_SIZE + 1
TWO_STD_SQ = 2.0 * (WINDOW_SIZE / 2.0) ** 2

_DOT = functools.partial(
    jax.lax.dot_general,
    preferred_element_type=jnp.float32,
)

_NW = 32
_WLP = 288
_RPW = 8 * _WLP // _NW


def _k0_body(dec_ref, w1_ref, b1_ref, w2_ref, b2_ref, len_ref, idx_ref):
    x = dec_ref[...]
    g = _DOT(x, w1_ref[...], (((1,), (1,)), ((), ())))
    t1 = jnp.tanh(g + b1_ref[...])
    z = _DOT(t1, w2_ref[...], (((1,), (1,)), ((), ())))[:, 0:1]
    sig = jax.nn.sigmoid(z + b2_ref[0])
    st = jnp.round(len_ref[...] * sig).astype(jnp.int32)
    B = st.shape[0]
    iw = jax.lax.broadcasted_iota(jnp.int32, (B, _WLP), 1)
    ib = jax.lax.broadcasted_iota(jnp.int32, (B, _WLP), 0)
    idx_ref[...] = jnp.where(iw < WL, (st + iw) * B + ib, 0)


def _sc_gather_body(idx_hbm, enc_hbm, out_hbm, idx_v, rows_v, sem):
    wid = lax.axis_index("s") * 2 + lax.axis_index("c")
    base = wid * _RPW
    pltpu.sync_copy(idx_hbm.at[pl.ds(base, _RPW)], idx_v)
    pltpu.async_copy(enc_hbm.at[idx_v], rows_v, sem).wait()
    pltpu.sync_copy(rows_v, out_hbm.at[pl.ds(base, _RPW)])


def _att_body(len_ref, dec_ref, sel_ref, w1_ref, b1_ref, w2_ref,
              b2_ref, out_ref, ht_v, out_v, sem_ht, sem_out):
    b = pl.program_id(0)
    nb = pl.num_programs(0)
    B = nb

    def ht_copy(bb):
        return pltpu.make_async_copy(
            dec_ref.at[:, bb, :], ht_v.at[bb], sem_ht.at[bb])

    def out_copy(bb):
        return pltpu.make_async_copy(
            out_v.at[bb % 2], out_ref.at[:, bb, :], sem_out.at[bb % 2])

    @pl.when(b == 0)
    def _():
        for bb in range(B):
            ht_copy(bb).start()

    ht_copy(b).wait()
    ht = ht_v[b]
    g = _DOT(ht, w1_ref[...], (((1,), (1,)), ((), ())))
    t1 = jnp.tanh(g + b1_ref[...])
    z = _DOT(t1, w2_ref[...], (((1,), (1,)), ((), ())))[:, 0:1]
    sig = jax.nn.sigmoid(z + b2_ref[0])
    length = len_ref[b].astype(jnp.float32)
    p = WINDOW_SIZE + length * sig
    ws = jnp.round(p - WINDOW_SIZE).astype(jnp.int32)

    T = ht.shape[0]
    iw = jax.lax.broadcasted_iota(jnp.int32, (T, WL), 1)
    pos = ws.astype(jnp.float32) + iw.astype(jnp.float32)
    gauss = jnp.exp(-((pos - p) ** 2) / TWO_STD_SQ)

    sel = sel_ref[0][:WL]
    score = _DOT(ht, sel, (((1,), (1,)), ((), ())))
    left = iw < (WINDOW_SIZE - ws)
    right = iw >= (len_ref[b] + WINDOW_SIZE - ws)
    score = jnp.where(left | right, jnp.float32(1e-14), score)
    m = jnp.max(score, axis=1, keepdims=True)
    e = jnp.exp(score - m)
    a = (e / jnp.sum(e, axis=1, keepdims=True)) * gauss

    @pl.when(b >= 2)
    def _():
        out_copy(b - 2).wait()

    c = _DOT(a[:, :WL - 1], sel[:WL - 1],
             (((1,), (0,)), ((), ())))
    c = c + a[:, WL - 1:WL] * sel[WL - 1:WL, :]
    out_v[b % 2] = c
    out_copy(b).start()

    @pl.when(b == nb - 1)
    def _():
        out_copy(b - 1).wait()
        out_copy(b).wait()


def kernel(encoder_output, decoder_output, lengths, fc1_w, fc1_b, fc2_w,
           fc2_b, T, batch_size, output_weights):
    S, B, H = encoder_output.shape
    Tn = decoder_output.shape[0]
    H2 = fc1_w.shape[0]

    b1 = fc1_b.reshape(1, H2)
    w2 = jnp.zeros((8, H2), jnp.float32).at[0].set(fc2_w.reshape(H2))
    b2 = fc2_b.reshape(1)
    len_f = lengths.astype(jnp.float32).reshape(B, 1)

    idx = pl.pallas_call(
        _k0_body,
        out_shape=jax.ShapeDtypeStruct((B, _WLP), jnp.int32),
        in_specs=[
            pl.BlockSpec(memory_space=pltpu.VMEM),
            pl.BlockSpec(memory_space=pltpu.VMEM),
            pl.BlockSpec(memory_space=pltpu.VMEM),
            pl.BlockSpec(memory_space=pltpu.VMEM),
            pl.BlockSpec(memory_space=pltpu.SMEM),
            pl.BlockSpec(memory_space=pltpu.VMEM),
        ],
        out_specs=pl.BlockSpec(memory_space=pltpu.VMEM),
    )(decoder_output[Tn - 1], fc1_w, b1, w2, b2, len_f)

    enc2d = encoder_output.reshape(S * B, H)

    sc_gather = functools.partial(
        pl.kernel,
        out_type=jax.ShapeDtypeStruct((B * _WLP, H), jnp.float32),
        mesh=plsc.VectorSubcoreMesh(core_axis_name="c", subcore_axis_name="s"),
        scratch_types=[
            pltpu.VMEM((_RPW,), jnp.int32),
            pltpu.VMEM((_RPW, H), jnp.float32),
            pltpu.SemaphoreType.DMA,
        ],
    )(_sc_gather_body)
    sel_flat = sc_gather(idx.reshape(B * _WLP), enc2d)
    selection = sel_flat.reshape(B, _WLP, H)

    out = pl.pallas_call(
        _att_body,
        grid=(B,),
        in_specs=[
            pl.BlockSpec(memory_space=pltpu.SMEM),
            pl.BlockSpec(memory_space=pl.ANY),
            pl.BlockSpec((1, _WLP, H), lambda b: (b, 0, 0)),
            pl.BlockSpec(memory_space=pltpu.VMEM),
            pl.BlockSpec(memory_space=pltpu.VMEM),
            pl.BlockSpec(memory_space=pltpu.VMEM),
            pl.BlockSpec(memory_space=pltpu.SMEM),
        ],
        out_specs=pl.BlockSpec(memory_space=pl.ANY),
        out_shape=jax.ShapeDtypeStruct((Tn, B, H), jnp.float32),
        scratch_shapes=[
            pltpu.VMEM((B, Tn, H), jnp.float32),
            pltpu.VMEM((2, Tn, H), jnp.float32),
            pltpu.SemaphoreType.DMA((B,)),
            pltpu.SemaphoreType.DMA((2,)),
        ],
    )(lengths, decoder_output, selection, fc1_w, b1, w2, b2)
    return out
